# Initial kernel scaffold; baseline (speedup 1.0000x reference)
import jax
import jax.numpy as jnp
from jax.experimental import pallas as pl


def kernel(f_atoms_0, f_bonds_0, a2b_0, b2a_0, b2revb_0, mol_ids_0, f_atoms_1, f_bonds_1, a2b_1, b2a_1, b2revb_1, mol_ids_1, W_i, W_h, W_o, b_o, w_ident, b_ident):
    raise NotImplementedError("write your pallas kernel here")


# speedup vs baseline: 1.2423x; 1.2423x over previous
import functools

import jax
import jax.numpy as jnp
from jax import lax
from jax.experimental import pallas as pl
from jax.experimental.pallas import tpu as pltpu
from jax.experimental.pallas import tpu_sc as plsc

H = 300
HP = 320
AF = 133
BF = 147
NA1 = 10000
NB1 = 40000
NAP = 20480
NBP = 81920
NM = 512
NW = 32

MB = 640

A_C = 16
A_CH = 40
A_IDX = A_C * 6

B_C = 64
B_CH = 40

_SL = HP // 32


_sc_cache = {}


def _nbr_sum(msg, idxa):
    if "nbr" not in _sc_cache:
        _sc_cache["nbr"] = _build_nbr_sum()
    return _sc_cache["nbr"](msg, idxa)


def _bond_msg(am, msg, b2a, b2revb):
    if "bond" not in _sc_cache:
        _sc_cache["bond"] = _build_bond_msg()
    return _sc_cache["bond"](am, msg, b2a, b2revb)


def _sc_mesh():
    return plsc.VectorSubcoreMesh(
        core_axis_name="c", subcore_axis_name="s", num_cores=2,
        num_subcores=16)


def _build_nbr_sum():
  @functools.partial(
      pl.kernel,
      out_type=jax.ShapeDtypeStruct((NAP, HP), jnp.bfloat16),
      mesh=_sc_mesh(),
      compiler_params=pltpu.CompilerParams(use_tc_tiling_on_sc=False),
      scratch_types=[
          pltpu.VMEM((A_CH * A_IDX,), jnp.int32),
          pltpu.VMEM((A_IDX, HP), jnp.bfloat16),
          pltpu.VMEM((A_IDX, HP), jnp.bfloat16),
          pltpu.VMEM((A_C, HP), jnp.bfloat16),
          pltpu.VMEM((A_C, HP), jnp.bfloat16),
          pltpu.SemaphoreType.DMA,
          pltpu.SemaphoreType.DMA,
          pltpu.SemaphoreType.DMA,
          pltpu.SemaphoreType.DMA,
      ],
  )
  def _nbr_sum_k(msg_hbm, idxa_hbm, out_hbm, idx_all,
                 buf0, buf1, acc0, acc1, gs0, gs1, ws0, ws1):
    wid = lax.axis_index("s") * 2 + lax.axis_index("c")
    pltpu.sync_copy(
        idxa_hbm.at[pl.ds(wid * A_CH * A_IDX, A_CH * A_IDX)], idx_all)
    pltpu.async_copy(
        msg_hbm.at[idx_all.at[pl.ds(0, A_IDX)]], buf0, gs0)
    pltpu.async_copy(
        msg_hbm.at[idx_all.at[pl.ds(A_IDX, A_IDX)]], buf1, gs1)

    def pair(c0, carry):
        for b, (buf, acc, gs, ws) in enumerate(
                ((buf0, acc0, gs0, ws0), (buf1, acc1, gs1, ws1))):
            c = 2 * c0 + b
            pltpu.make_async_copy(
                msg_hbm.at[idx_all.at[pl.ds(0, A_IDX)]], buf, gs).wait()

            @pl.when(c0 > 0)
            def _():
                pltpu.make_async_copy(
                    acc, out_hbm.at[pl.ds(0, A_C)], ws).wait()

            def body(a, carry2):
                for u in range(_SL):
                    s = pl.ds(u * 32, 32)
                    acc[a, s] = (
                        buf[a, s]
                        + buf[A_C + a, s]
                        + buf[2 * A_C + a, s]
                        + buf[3 * A_C + a, s]
                        + buf[4 * A_C + a, s]
                        + buf[5 * A_C + a, s]
                    )
                return carry2

            lax.fori_loop(0, A_C, body, 0)
            pltpu.async_copy(
                acc, out_hbm.at[pl.ds((wid * A_CH + c) * A_C, A_C)], ws)

            @pl.when(c + 2 < A_CH)
            def _():
                pltpu.async_copy(
                    msg_hbm.at[idx_all.at[pl.ds((c + 2) * A_IDX, A_IDX)]],
                    buf, gs)
        return carry

    lax.fori_loop(0, A_CH // 2, pair, 0)
    pltpu.make_async_copy(acc0, out_hbm.at[pl.ds(0, A_C)], ws0).wait()
    pltpu.make_async_copy(acc1, out_hbm.at[pl.ds(0, A_C)], ws1).wait()

  return _nbr_sum_k


def _build_bond_msg():
  @functools.partial(
      pl.kernel,
      out_type=jax.ShapeDtypeStruct((NBP, HP), jnp.bfloat16),
      mesh=_sc_mesh(),
      compiler_params=pltpu.CompilerParams(use_tc_tiling_on_sc=False),
      scratch_types=[
          pltpu.VMEM((B_CH * B_C,), jnp.int32),
          pltpu.VMEM((B_CH * B_C,), jnp.int32),
          pltpu.VMEM((B_C, HP), jnp.bfloat16),
          pltpu.VMEM((B_C, HP), jnp.bfloat16),
          pltpu.VMEM((B_C, HP), jnp.bfloat16),
          pltpu.VMEM((B_C, HP), jnp.bfloat16),
          pltpu.VMEM((B_C, HP), jnp.bfloat16),
          pltpu.VMEM((B_C, HP), jnp.bfloat16),
          pltpu.SemaphoreType.DMA,
          pltpu.SemaphoreType.DMA,
          pltpu.SemaphoreType.DMA,
          pltpu.SemaphoreType.DMA,
          pltpu.SemaphoreType.DMA,
          pltpu.SemaphoreType.DMA,
      ],
  )
  def _bond_msg_k(am_hbm, msg_hbm, b2a_hbm, b2revb_hbm, out_hbm,
                  i1_all, i2_all, ba0, ba1, bb0, bb1, ob0, ob1,
                  ga0, ga1, gb0, gb1, ws0, ws1):
    wid = lax.axis_index("s") * 2 + lax.axis_index("c")
    npw = B_CH * B_C
    pltpu.sync_copy(b2a_hbm.at[pl.ds(wid * npw, npw)], i1_all)
    pltpu.sync_copy(b2revb_hbm.at[pl.ds(wid * npw, npw)], i2_all)
    pltpu.async_copy(am_hbm.at[i1_all.at[pl.ds(0, B_C)]], ba0, ga0)
    pltpu.async_copy(msg_hbm.at[i2_all.at[pl.ds(0, B_C)]], bb0, gb0)
    pltpu.async_copy(am_hbm.at[i1_all.at[pl.ds(B_C, B_C)]], ba1, ga1)
    pltpu.async_copy(msg_hbm.at[i2_all.at[pl.ds(B_C, B_C)]], bb1, gb1)

    def pair(c0, carry):
        for b, (ba, bb, ob, ga, gb, ws) in enumerate(
                ((ba0, bb0, ob0, ga0, gb0, ws0),
                 (ba1, bb1, ob1, ga1, gb1, ws1))):
            c = 2 * c0 + b
            pltpu.make_async_copy(
                am_hbm.at[i1_all.at[pl.ds(0, B_C)]], ba, ga).wait()
            pltpu.make_async_copy(
                msg_hbm.at[i2_all.at[pl.ds(0, B_C)]], bb, gb).wait()

            @pl.when(c0 > 0)
            def _():
                pltpu.make_async_copy(
                    ob, out_hbm.at[pl.ds(0, B_C)], ws).wait()

            def body(i, carry2):
                for u in range(_SL):
                    s = pl.ds(u * 32, 32)
                    ob[i, s] = ba[i, s] - bb[i, s]
                return carry2

            lax.fori_loop(0, B_C, body, 0)

            @pl.when(c + 2 < B_CH)
            def _():
                pltpu.async_copy(
                    am_hbm.at[i1_all.at[pl.ds((c + 2) * B_C, B_C)]], ba, ga)
                pltpu.async_copy(
                    msg_hbm.at[i2_all.at[pl.ds((c + 2) * B_C, B_C)]], bb, gb)

            pltpu.async_copy(
                ob, out_hbm.at[pl.ds(wid * npw + c * B_C, B_C)], ws)
        return carry

    lax.fori_loop(0, B_CH // 2, pair, 0)
    pltpu.make_async_copy(ob0, out_hbm.at[pl.ds(0, B_C)], ws0).wait()
    pltpu.make_async_copy(ob1, out_hbm.at[pl.ds(0, B_C)], ws1).wait()

  return _bond_msg_k


def _k_in_body(x_ref, w_ref, inp_ref, msg_ref):
    acc = jnp.dot(x_ref[...], w_ref[...], preferred_element_type=jnp.float32)
    inp_ref[...] = acc.astype(jnp.bfloat16)
    msg_ref[...] = jnp.maximum(acc, 0.0).astype(jnp.bfloat16)


def _k_in(fb, w):
    grid = NBP // MB
    return pl.pallas_call(
        _k_in_body,
        grid=(grid,),
        in_specs=[
            pl.BlockSpec((MB, BF), lambda i: (i, 0)),
            pl.BlockSpec((BF, HP), lambda i: (0, 0)),
        ],
        out_specs=[
            pl.BlockSpec((MB, HP), lambda i: (i, 0)),
            pl.BlockSpec((MB, HP), lambda i: (i, 0)),
        ],
        out_shape=[
            jax.ShapeDtypeStruct((NBP, HP), jnp.bfloat16),
            jax.ShapeDtypeStruct((NBP, HP), jnp.bfloat16),
        ],
    )(fb, w)


def _k_upd_body(m2_ref, inp_ref, w_ref, msg_ref):
    acc = jnp.dot(m2_ref[...], w_ref[...], preferred_element_type=jnp.float32)
    acc += inp_ref[...].astype(jnp.float32)
    msg_ref[...] = jnp.maximum(acc, 0.0).astype(jnp.bfloat16)


def _k_upd(m2, inp, w):
    grid = NBP // MB
    return pl.pallas_call(
        _k_upd_body,
        grid=(grid,),
        in_specs=[
            pl.BlockSpec((MB, HP), lambda i: (i, 0)),
            pl.BlockSpec((MB, HP), lambda i: (i, 0)),
            pl.BlockSpec((HP, HP), lambda i: (0, 0)),
        ],
        out_specs=pl.BlockSpec((MB, HP), lambda i: (i, 0)),
        out_shape=jax.ShapeDtypeStruct((NBP, HP), jnp.bfloat16),
    )(m2, inp, w)


def _k_out_body(fa_ref, am_ref, w1_ref, w2_ref, b_ref, out_ref):
    acc = jnp.dot(fa_ref[...], w1_ref[...], preferred_element_type=jnp.float32)
    acc += jnp.dot(am_ref[...], w2_ref[...], preferred_element_type=jnp.float32)
    out_ref[...] = jnp.maximum(acc + b_ref[...], 0.0)


def _k_out(fa, am, w1, w2, b):
    grid = NAP // MB
    return pl.pallas_call(
        _k_out_body,
        grid=(grid,),
        in_specs=[
            pl.BlockSpec((MB, AF), lambda i: (i, 0)),
            pl.BlockSpec((MB, HP), lambda i: (i, 0)),
            pl.BlockSpec((AF, HP), lambda i: (0, 0)),
            pl.BlockSpec((HP, HP), lambda i: (0, 0)),
            pl.BlockSpec((1, HP), lambda i: (0, 0)),
        ],
        out_specs=pl.BlockSpec((MB, HP), lambda i: (i, 0)),
        out_shape=jax.ShapeDtypeStruct((NAP, HP), jnp.float32),
    )(fa, am, w1, w2, b)


def _k_read_body(ah_ref, ids_ref, wv_ref, out_ref, sums_ref, cnts_ref):
    i = pl.program_id(0)

    @pl.when(i == 0)
    def _():
        sums_ref[...] = jnp.zeros_like(sums_ref)
        cnts_ref[...] = jnp.zeros_like(cnts_ref)

    ids = ids_ref[0]
    seg = lax.broadcasted_iota(jnp.int32, (NM, MB), 0)
    onehot = (seg == jnp.broadcast_to(ids, (NM, MB))).astype(jnp.float32)
    sums_ref[...] += jnp.dot(onehot, ah_ref[...],
                             preferred_element_type=jnp.float32)
    cnts_ref[...] += jnp.broadcast_to(
        jnp.sum(onehot, axis=1, keepdims=True), (NM, 128))

    @pl.when(i == pl.num_programs(0) - 1)
    def _():
        enc = sums_ref[...] / jnp.maximum(cnts_ref[:, 0:1], 1.0)
        score = jnp.dot(enc, wv_ref[...], preferred_element_type=jnp.float32)
        out_ref[...] = jax.nn.sigmoid(score[:NM // 2] - score[NM // 2:])


def _k_read(ah, ids3, wv):
    grid = NAP // MB
    return pl.pallas_call(
        _k_read_body,
        grid=(grid,),
        in_specs=[
            pl.BlockSpec((MB, HP), lambda i: (i, 0)),
            pl.BlockSpec((1, 1, MB), lambda i: (i, 0, 0)),
            pl.BlockSpec((HP, 1), lambda i: (0, 0)),
        ],
        out_specs=pl.BlockSpec((NM // 2, 1), lambda i: (0, 0)),
        out_shape=jax.ShapeDtypeStruct((NM // 2, 1), jnp.float32),
        scratch_shapes=[
            pltpu.VMEM((NM, HP), jnp.float32),
            pltpu.VMEM((NM, 128), jnp.float32),
        ],
    )(ah, ids3, wv)


def kernel(f_atoms_0, f_bonds_0, a2b_0, b2a_0, b2revb_0, mol_ids_0,
           f_atoms_1, f_bonds_1, a2b_1, b2a_1, b2revb_1, mol_ids_1,
           W_i, W_h, W_o, b_o, w_ident, b_ident):
    f32 = jnp.float32
    bf16 = jnp.bfloat16

    fb = jnp.concatenate(
        [f_bonds_0.astype(bf16), f_bonds_1.astype(bf16),
         jnp.zeros((NBP - 2 * NB1, BF), bf16)], axis=0)
    fa = jnp.concatenate(
        [f_atoms_0.astype(bf16), f_atoms_1.astype(bf16),
         jnp.zeros((NAP - 2 * NA1, AF), bf16)], axis=0)
    a2b = jnp.concatenate(
        [a2b_0, a2b_1 + NB1,
         jnp.zeros((NAP - 2 * NA1, 6), jnp.int32)], axis=0)
    b2a = jnp.concatenate(
        [b2a_0, b2a_1 + NA1, jnp.zeros((NBP - 2 * NB1,), jnp.int32)], axis=0)
    b2revb = jnp.concatenate(
        [b2revb_0, b2revb_1 + NB1,
         jnp.zeros((NBP - 2 * NB1,), jnp.int32)], axis=0)
    mids = jnp.concatenate(
        [mol_ids_0, mol_ids_1 + NM // 2,
         jnp.full((NAP - 2 * NA1,), NM, jnp.int32)], axis=0)
    ids3 = mids.reshape(NAP // MB, 1, MB)

    idxa = (a2b.reshape(NW, A_CH, A_C, 6)
               .transpose(0, 1, 3, 2)
               .reshape(NW * A_CH * A_IDX))

    wi = jnp.zeros((BF, HP), f32).at[:, :H].set(W_i).astype(bf16)
    wh = jnp.zeros((HP, HP), f32).at[:H, :H].set(W_h).astype(bf16)
    wo1 = jnp.zeros((AF, HP), f32).at[:, :H].set(W_o[:AF]).astype(bf16)
    wo2 = jnp.zeros((HP, HP), f32).at[:H, :H].set(W_o[AF:]).astype(bf16)
    bo = jnp.zeros((1, HP), f32).at[0, :H].set(b_o)
    wv = jnp.zeros((HP, 1), f32).at[:H].set(w_ident)

    inp, msg = _k_in(fb, wi)
    for _ in range(2):
        am = _nbr_sum(msg, idxa)
        m2 = _bond_msg(am, msg, b2a, b2revb)
        msg = _k_upd(m2, inp, wh)
    am = _nbr_sum(msg, idxa)
    ah = _k_out(fa, am, wo1, wo2, bo)
    out = _k_read(ah, ids3, wv)
    return out[:, 0]


# speedup vs baseline: 1.6828x; 1.3546x over previous
import functools

import jax
import jax.numpy as jnp
from jax import lax
from jax.experimental import pallas as pl
from jax.experimental.pallas import tpu as pltpu
from jax.experimental.pallas import tpu_sc as plsc

H = 300
HP = 384
AF = 133
BF = 147
NA1 = 10000
NB1 = 40000
NAP = 20480
NBP = 81920
NM = 512
NW = 32

MB = 640

A_C = 16
A_CH = 40
A_IDX = A_C * 6

B_C = 40
B_CH = 64

_SL = HP // 16


_sc_cache = {}


def _nbr_sum(msg, idxa):
    if "nbr" not in _sc_cache:
        _sc_cache["nbr"] = _build_nbr_sum()
    return _sc_cache["nbr"](msg, idxa)


def _bond_msg(am, msg, b2a, b2revb):
    if "bond" not in _sc_cache:
        _sc_cache["bond"] = _build_bond_msg()
    return _sc_cache["bond"](am, msg, b2a, b2revb)


def _sc_mesh():
    return plsc.VectorSubcoreMesh(
        core_axis_name="c", subcore_axis_name="s", num_cores=2,
        num_subcores=16)


def _build_nbr_sum():
  @functools.partial(
      pl.kernel,
      out_type=jax.ShapeDtypeStruct((NAP, HP), jnp.float32),
      mesh=_sc_mesh(),
      scratch_types=[
          pltpu.VMEM((A_CH * A_IDX,), jnp.int32),
          pltpu.VMEM((A_IDX, HP), jnp.float32),
          pltpu.VMEM((A_IDX, HP), jnp.float32),
          pltpu.VMEM((A_C, HP), jnp.float32),
          pltpu.VMEM((A_C, HP), jnp.float32),
          pltpu.SemaphoreType.DMA,
          pltpu.SemaphoreType.DMA,
          pltpu.SemaphoreType.DMA,
          pltpu.SemaphoreType.DMA,
      ],
  )
  def _nbr_sum_k(msg_hbm, idxa_hbm, out_hbm, idx_all,
                 buf0, buf1, acc0, acc1, gs0, gs1, ws0, ws1):
    wid = lax.axis_index("s") * 2 + lax.axis_index("c")
    pltpu.sync_copy(
        idxa_hbm.at[pl.ds(wid * A_CH * A_IDX, A_CH * A_IDX)], idx_all)
    pltpu.async_copy(
        msg_hbm.at[idx_all.at[pl.ds(0, A_IDX)]], buf0, gs0)
    pltpu.async_copy(
        msg_hbm.at[idx_all.at[pl.ds(A_IDX, A_IDX)]], buf1, gs1)

    def pair(c0, carry):
        for b, (buf, acc, gs, ws) in enumerate(
                ((buf0, acc0, gs0, ws0), (buf1, acc1, gs1, ws1))):
            c = 2 * c0 + b
            pltpu.make_async_copy(
                msg_hbm.at[idx_all.at[pl.ds(0, A_IDX)]], buf, gs).wait()

            @pl.when(c0 > 0)
            def _():
                pltpu.make_async_copy(
                    acc, out_hbm.at[pl.ds(0, A_C)], ws).wait()

            def body(a, carry2):
                for u in range(_SL):
                    s = pl.ds(u * 16, 16)
                    acc[a, s] = (
                        buf[a, s]
                        + buf[A_C + a, s]
                        + buf[2 * A_C + a, s]
                        + buf[3 * A_C + a, s]
                        + buf[4 * A_C + a, s]
                        + buf[5 * A_C + a, s]
                    )
                return carry2

            lax.fori_loop(0, A_C, body, 0)
            pltpu.async_copy(
                acc, out_hbm.at[pl.ds((wid * A_CH + c) * A_C, A_C)], ws)

            @pl.when(c + 2 < A_CH)
            def _():
                pltpu.async_copy(
                    msg_hbm.at[idx_all.at[pl.ds((c + 2) * A_IDX, A_IDX)]],
                    buf, gs)
        return carry

    lax.fori_loop(0, A_CH // 2, pair, 0)
    pltpu.make_async_copy(acc0, out_hbm.at[pl.ds(0, A_C)], ws0).wait()
    pltpu.make_async_copy(acc1, out_hbm.at[pl.ds(0, A_C)], ws1).wait()

  return _nbr_sum_k


def _build_bond_msg():
  @functools.partial(
      pl.kernel,
      out_type=jax.ShapeDtypeStruct((NBP, HP), jnp.float32),
      mesh=_sc_mesh(),
      scratch_types=[
          pltpu.VMEM((B_CH * B_C,), jnp.int32),
          pltpu.VMEM((B_CH * B_C,), jnp.int32),
          pltpu.VMEM((B_C, HP), jnp.float32),
          pltpu.VMEM((B_C, HP), jnp.float32),
          pltpu.VMEM((B_C, HP), jnp.float32),
          pltpu.VMEM((B_C, HP), jnp.float32),
          pltpu.VMEM((B_C, HP), jnp.float32),
          pltpu.VMEM((B_C, HP), jnp.float32),
          pltpu.SemaphoreType.DMA,
          pltpu.SemaphoreType.DMA,
          pltpu.SemaphoreType.DMA,
          pltpu.SemaphoreType.DMA,
          pltpu.SemaphoreType.DMA,
          pltpu.SemaphoreType.DMA,
      ],
  )
  def _bond_msg_k(am_hbm, msg_hbm, b2a_hbm, b2revb_hbm, out_hbm,
                  i1_all, i2_all, ba0, ba1, bb0, bb1, ob0, ob1,
                  ga0, ga1, gb0, gb1, ws0, ws1):
    wid = lax.axis_index("s") * 2 + lax.axis_index("c")
    npw = B_CH * B_C
    pltpu.sync_copy(b2a_hbm.at[pl.ds(wid * npw, npw)], i1_all)
    pltpu.sync_copy(b2revb_hbm.at[pl.ds(wid * npw, npw)], i2_all)
    pltpu.async_copy(am_hbm.at[i1_all.at[pl.ds(0, B_C)]], ba0, ga0)
    pltpu.async_copy(msg_hbm.at[i2_all.at[pl.ds(0, B_C)]], bb0, gb0)
    pltpu.async_copy(am_hbm.at[i1_all.at[pl.ds(B_C, B_C)]], ba1, ga1)
    pltpu.async_copy(msg_hbm.at[i2_all.at[pl.ds(B_C, B_C)]], bb1, gb1)

    def pair(c0, carry):
        for b, (ba, bb, ob, ga, gb, ws) in enumerate(
                ((ba0, bb0, ob0, ga0, gb0, ws0),
                 (ba1, bb1, ob1, ga1, gb1, ws1))):
            c = 2 * c0 + b
            pltpu.make_async_copy(
                am_hbm.at[i1_all.at[pl.ds(0, B_C)]], ba, ga).wait()
            pltpu.make_async_copy(
                msg_hbm.at[i2_all.at[pl.ds(0, B_C)]], bb, gb).wait()

            @pl.when(c0 > 0)
            def _():
                pltpu.make_async_copy(
                    ob, out_hbm.at[pl.ds(0, B_C)], ws).wait()

            def body(i, carry2):
                for u in range(_SL):
                    s = pl.ds(u * 16, 16)
                    ob[i, s] = ba[i, s] - bb[i, s]
                return carry2

            lax.fori_loop(0, B_C, body, 0)

            @pl.when(c + 2 < B_CH)
            def _():
                pltpu.async_copy(
                    am_hbm.at[i1_all.at[pl.ds((c + 2) * B_C, B_C)]], ba, ga)
                pltpu.async_copy(
                    msg_hbm.at[i2_all.at[pl.ds((c + 2) * B_C, B_C)]], bb, gb)

            pltpu.async_copy(
                ob, out_hbm.at[pl.ds(wid * npw + c * B_C, B_C)], ws)
        return carry

    lax.fori_loop(0, B_CH // 2, pair, 0)
    pltpu.make_async_copy(ob0, out_hbm.at[pl.ds(0, B_C)], ws0).wait()
    pltpu.make_async_copy(ob1, out_hbm.at[pl.ds(0, B_C)], ws1).wait()

  return _bond_msg_k


def _k_in_body(x_ref, w_ref, inp_ref, msg_ref):
    acc = jnp.dot(x_ref[...], w_ref[...], preferred_element_type=jnp.float32)
    inp_ref[...] = acc.astype(jnp.bfloat16)
    msg_ref[...] = jnp.maximum(acc, 0.0)


def _k_in(fb, w):
    grid = NBP // MB
    return pl.pallas_call(
        _k_in_body,
        grid=(grid,),
        in_specs=[
            pl.BlockSpec((MB, BF), lambda i: (i, 0)),
            pl.BlockSpec((BF, HP), lambda i: (0, 0)),
        ],
        out_specs=[
            pl.BlockSpec((MB, HP), lambda i: (i, 0)),
            pl.BlockSpec((MB, HP), lambda i: (i, 0)),
        ],
        out_shape=[
            jax.ShapeDtypeStruct((NBP, HP), jnp.bfloat16),
            jax.ShapeDtypeStruct((NBP, HP), jnp.float32),
        ],
    )(fb, w)


def _k_upd_body(m2_ref, inp_ref, w_ref, msg_ref):
    acc = jnp.dot(m2_ref[...].astype(jnp.bfloat16), w_ref[...],
                  preferred_element_type=jnp.float32)
    acc += inp_ref[...].astype(jnp.float32)
    msg_ref[...] = jnp.maximum(acc, 0.0)


def _k_upd(m2, inp, w):
    grid = NBP // MB
    return pl.pallas_call(
        _k_upd_body,
        grid=(grid,),
        in_specs=[
            pl.BlockSpec((MB, HP), lambda i: (i, 0)),
            pl.BlockSpec((MB, HP), lambda i: (i, 0)),
            pl.BlockSpec((HP, HP), lambda i: (0, 0)),
        ],
        out_specs=pl.BlockSpec((MB, HP), lambda i: (i, 0)),
        out_shape=jax.ShapeDtypeStruct((NBP, HP), jnp.float32),
    )(m2, inp, w)


def _k_out_body(fa_ref, am_ref, w1_ref, w2_ref, b_ref, out_ref):
    acc = jnp.dot(fa_ref[...], w1_ref[...], preferred_element_type=jnp.float32)
    acc += jnp.dot(am_ref[...].astype(jnp.bfloat16), w2_ref[...],
                   preferred_element_type=jnp.float32)
    out_ref[...] = jnp.maximum(acc + b_ref[...], 0.0)


def _k_out(fa, am, w1, w2, b):
    grid = NAP // MB
    return pl.pallas_call(
        _k_out_body,
        grid=(grid,),
        in_specs=[
            pl.BlockSpec((MB, AF), lambda i: (i, 0)),
            pl.BlockSpec((MB, HP), lambda i: (i, 0)),
            pl.BlockSpec((AF, HP), lambda i: (0, 0)),
            pl.BlockSpec((HP, HP), lambda i: (0, 0)),
            pl.BlockSpec((1, HP), lambda i: (0, 0)),
        ],
        out_specs=pl.BlockSpec((MB, HP), lambda i: (i, 0)),
        out_shape=jax.ShapeDtypeStruct((NAP, HP), jnp.float32),
    )(fa, am, w1, w2, b)


def _k_read_body(ah_ref, ids_ref, wv_ref, out_ref, sums_ref, cnts_ref):
    i = pl.program_id(0)

    @pl.when(i == 0)
    def _():
        sums_ref[...] = jnp.zeros_like(sums_ref)
        cnts_ref[...] = jnp.zeros_like(cnts_ref)

    ids = ids_ref[0]
    seg = lax.broadcasted_iota(jnp.int32, (NM, MB), 0)
    onehot = (seg == jnp.broadcast_to(ids, (NM, MB))).astype(jnp.float32)
    sums_ref[...] += jnp.dot(onehot, ah_ref[...],
                             preferred_element_type=jnp.float32)
    cnts_ref[...] += jnp.broadcast_to(
        jnp.sum(onehot, axis=1, keepdims=True), (NM, 128))

    @pl.when(i == pl.num_programs(0) - 1)
    def _():
        enc = sums_ref[...] / jnp.maximum(cnts_ref[:, 0:1], 1.0)
        score = jnp.dot(enc, wv_ref[...], preferred_element_type=jnp.float32)
        out_ref[...] = jax.nn.sigmoid(score[:NM // 2] - score[NM // 2:])


def _k_read(ah, ids3, wv):
    grid = NAP // MB
    return pl.pallas_call(
        _k_read_body,
        grid=(grid,),
        in_specs=[
            pl.BlockSpec((MB, HP), lambda i: (i, 0)),
            pl.BlockSpec((1, 1, MB), lambda i: (i, 0, 0)),
            pl.BlockSpec((HP, 1), lambda i: (0, 0)),
        ],
        out_specs=pl.BlockSpec((NM // 2, 1), lambda i: (0, 0)),
        out_shape=jax.ShapeDtypeStruct((NM // 2, 1), jnp.float32),
        scratch_shapes=[
            pltpu.VMEM((NM, HP), jnp.float32),
            pltpu.VMEM((NM, 128), jnp.float32),
        ],
    )(ah, ids3, wv)


def kernel(f_atoms_0, f_bonds_0, a2b_0, b2a_0, b2revb_0, mol_ids_0,
           f_atoms_1, f_bonds_1, a2b_1, b2a_1, b2revb_1, mol_ids_1,
           W_i, W_h, W_o, b_o, w_ident, b_ident):
    f32 = jnp.float32
    bf16 = jnp.bfloat16

    fb = jnp.concatenate(
        [f_bonds_0.astype(bf16), f_bonds_1.astype(bf16),
         jnp.zeros((NBP - 2 * NB1, BF), bf16)], axis=0)
    fa = jnp.concatenate(
        [f_atoms_0.astype(bf16), f_atoms_1.astype(bf16),
         jnp.zeros((NAP - 2 * NA1, AF), bf16)], axis=0)
    a2b = jnp.concatenate(
        [a2b_0, a2b_1 + NB1,
         jnp.zeros((NAP - 2 * NA1, 6), jnp.int32)], axis=0)
    b2a = jnp.concatenate(
        [b2a_0, b2a_1 + NA1, jnp.zeros((NBP - 2 * NB1,), jnp.int32)], axis=0)
    b2revb = jnp.concatenate(
        [b2revb_0, b2revb_1 + NB1,
         jnp.zeros((NBP - 2 * NB1,), jnp.int32)], axis=0)
    mids = jnp.concatenate(
        [mol_ids_0, mol_ids_1 + NM // 2,
         jnp.full((NAP - 2 * NA1,), NM, jnp.int32)], axis=0)
    ids3 = mids.reshape(NAP // MB, 1, MB)

    idxa = (a2b.reshape(NW, A_CH, A_C, 6)
               .transpose(0, 1, 3, 2)
               .reshape(NW * A_CH * A_IDX))

    wi = jnp.zeros((BF, HP), f32).at[:, :H].set(W_i).astype(bf16)
    wh = jnp.zeros((HP, HP), f32).at[:H, :H].set(W_h).astype(bf16)
    wo1 = jnp.zeros((AF, HP), f32).at[:, :H].set(W_o[:AF]).astype(bf16)
    wo2 = jnp.zeros((HP, HP), f32).at[:H, :H].set(W_o[AF:]).astype(bf16)
    bo = jnp.zeros((1, HP), f32).at[0, :H].set(b_o)
    wv = jnp.zeros((HP, 1), f32).at[:H].set(w_ident)

    inp, msg = _k_in(fb, wi)
    for _ in range(2):
        am = _nbr_sum(msg, idxa)
        m2 = _bond_msg(am, msg, b2a, b2revb)
        msg = _k_upd(m2, inp, wh)
    am = _nbr_sum(msg, idxa)
    ah = _k_out(fa, am, wo1, wo2, bo)
    out = _k_read(ah, ids3, wv)
    return out[:, 0]


# speedup vs baseline: 1.7214x; 1.0230x over previous
import functools

import jax
import jax.numpy as jnp
from jax import lax
from jax.experimental import pallas as pl
from jax.experimental.pallas import tpu as pltpu
from jax.experimental.pallas import tpu_sc as plsc

H = 300
HP = 384
AF = 133
BF = 147
NA1 = 10000
NB1 = 40000
NAP = 20480
NBP = 81920
NM = 512
NW = 32

MB = 640

A_C = 16
A_CH = 40
A_IDX = A_C * 6

B_C = 40
B_CH = 64

NC0A = 56
NC1A = 24
NC0B = 78
NC1B = 50

_SL = HP // 16


_sc_cache = {}


def _nbr_sum(msg, idxa):
    if "nbr" not in _sc_cache:
        _sc_cache["nbr"] = _build_nbr_sum()
    return _sc_cache["nbr"](msg, idxa)


def _bond_msg(am, msg, b2a, b2revb):
    if "bond" not in _sc_cache:
        _sc_cache["bond"] = _build_bond_msg()
    return _sc_cache["bond"](am, msg, b2a, b2revb)


def _sc_mesh():
    return plsc.VectorSubcoreMesh(
        core_axis_name="c", subcore_axis_name="s", num_cores=2,
        num_subcores=16)


def _build_nbr_sum():
  @functools.partial(
      pl.kernel,
      out_type=jax.ShapeDtypeStruct((NAP, HP), jnp.float32),
      mesh=_sc_mesh(),
      scratch_types=[
          pltpu.VMEM((NC0A * A_IDX,), jnp.int32),
          pltpu.VMEM((A_IDX, HP), jnp.float32),
          pltpu.VMEM((A_IDX, HP), jnp.float32),
          pltpu.VMEM((A_C, HP), jnp.float32),
          pltpu.VMEM((A_C, HP), jnp.float32),
          pltpu.SemaphoreType.DMA,
          pltpu.SemaphoreType.DMA,
          pltpu.SemaphoreType.DMA,
          pltpu.SemaphoreType.DMA,
      ],
  )
  def _nbr_sum_k(msg_hbm, idxa_hbm, out_hbm, idx_all,
                 buf0, buf1, acc0, acc1, gs0, gs1, ws0, ws1):
    core = lax.axis_index("c")
    sub = lax.axis_index("s")
    g0 = jnp.where(core == 0, sub * NC0A, 16 * NC0A + sub * NC1A)
    n_pairs = jnp.where(core == 0, NC0A // 2, NC1A // 2)
    n_ch = 2 * n_pairs
    pltpu.sync_copy(
        idxa_hbm.at[pl.ds(g0 * A_IDX, NC0A * A_IDX)], idx_all)
    pltpu.async_copy(
        msg_hbm.at[idx_all.at[pl.ds(0, A_IDX)]], buf0, gs0)
    pltpu.async_copy(
        msg_hbm.at[idx_all.at[pl.ds(A_IDX, A_IDX)]], buf1, gs1)

    def pair(c0, carry):
        for b, (buf, acc, gs, ws) in enumerate(
                ((buf0, acc0, gs0, ws0), (buf1, acc1, gs1, ws1))):
            c = 2 * c0 + b
            pltpu.make_async_copy(
                msg_hbm.at[idx_all.at[pl.ds(0, A_IDX)]], buf, gs).wait()

            @pl.when(c0 > 0)
            def _():
                pltpu.make_async_copy(
                    acc, out_hbm.at[pl.ds(0, A_C)], ws).wait()

            def body(a, carry2):
                for u in range(_SL):
                    s = pl.ds(u * 16, 16)
                    acc[a, s] = (
                        buf[a, s]
                        + buf[A_C + a, s]
                        + buf[2 * A_C + a, s]
                        + buf[3 * A_C + a, s]
                        + buf[4 * A_C + a, s]
                        + buf[5 * A_C + a, s]
                    )
                return carry2

            lax.fori_loop(0, A_C, body, 0)
            pltpu.async_copy(
                acc, out_hbm.at[pl.ds((g0 + c) * A_C, A_C)], ws)

            @pl.when(c + 2 < n_ch)
            def _():
                pltpu.async_copy(
                    msg_hbm.at[idx_all.at[pl.ds((c + 2) * A_IDX, A_IDX)]],
                    buf, gs)
        return carry

    lax.fori_loop(0, n_pairs, pair, 0)
    pltpu.make_async_copy(acc0, out_hbm.at[pl.ds(0, A_C)], ws0).wait()
    pltpu.make_async_copy(acc1, out_hbm.at[pl.ds(0, A_C)], ws1).wait()

  return _nbr_sum_k


def _build_bond_msg():
  @functools.partial(
      pl.kernel,
      out_type=jax.ShapeDtypeStruct((NBP, HP), jnp.float32),
      mesh=_sc_mesh(),
      scratch_types=[
          pltpu.VMEM((NC0B * B_C,), jnp.int32),
          pltpu.VMEM((NC0B * B_C,), jnp.int32),
          pltpu.VMEM((B_C, HP), jnp.float32),
          pltpu.VMEM((B_C, HP), jnp.float32),
          pltpu.VMEM((B_C, HP), jnp.float32),
          pltpu.VMEM((B_C, HP), jnp.float32),
          pltpu.VMEM((B_C, HP), jnp.float32),
          pltpu.VMEM((B_C, HP), jnp.float32),
          pltpu.SemaphoreType.DMA,
          pltpu.SemaphoreType.DMA,
          pltpu.SemaphoreType.DMA,
          pltpu.SemaphoreType.DMA,
          pltpu.SemaphoreType.DMA,
          pltpu.SemaphoreType.DMA,
      ],
  )
  def _bond_msg_k(am_hbm, msg_hbm, b2a_hbm, b2revb_hbm, out_hbm,
                  i1_all, i2_all, ba0, ba1, bb0, bb1, ob0, ob1,
                  ga0, ga1, gb0, gb1, ws0, ws1):
    core = lax.axis_index("c")
    sub = lax.axis_index("s")
    g0 = jnp.where(core == 0, sub * NC0B, 16 * NC0B + sub * NC1B)
    n_pairs = jnp.where(core == 0, NC0B // 2, NC1B // 2)
    n_ch = 2 * n_pairs
    pltpu.sync_copy(b2a_hbm.at[pl.ds(g0 * B_C, NC0B * B_C)], i1_all)
    pltpu.sync_copy(b2revb_hbm.at[pl.ds(g0 * B_C, NC0B * B_C)], i2_all)
    pltpu.async_copy(am_hbm.at[i1_all.at[pl.ds(0, B_C)]], ba0, ga0)
    pltpu.async_copy(msg_hbm.at[i2_all.at[pl.ds(0, B_C)]], bb0, gb0)
    pltpu.async_copy(am_hbm.at[i1_all.at[pl.ds(B_C, B_C)]], ba1, ga1)
    pltpu.async_copy(msg_hbm.at[i2_all.at[pl.ds(B_C, B_C)]], bb1, gb1)

    def pair(c0, carry):
        for b, (ba, bb, ob, ga, gb, ws) in enumerate(
                ((ba0, bb0, ob0, ga0, gb0, ws0),
                 (ba1, bb1, ob1, ga1, gb1, ws1))):
            c = 2 * c0 + b
            pltpu.make_async_copy(
                am_hbm.at[i1_all.at[pl.ds(0, B_C)]], ba, ga).wait()
            pltpu.make_async_copy(
                msg_hbm.at[i2_all.at[pl.ds(0, B_C)]], bb, gb).wait()

            @pl.when(c0 > 0)
            def _():
                pltpu.make_async_copy(
                    ob, out_hbm.at[pl.ds(0, B_C)], ws).wait()

            def body(i, carry2):
                for u in range(_SL):
                    s = pl.ds(u * 16, 16)
                    ob[i, s] = ba[i, s] - bb[i, s]
                return carry2

            lax.fori_loop(0, B_C, body, 0)

            @pl.when(c + 2 < n_ch)
            def _():
                pltpu.async_copy(
                    am_hbm.at[i1_all.at[pl.ds((c + 2) * B_C, B_C)]], ba, ga)
                pltpu.async_copy(
                    msg_hbm.at[i2_all.at[pl.ds((c + 2) * B_C, B_C)]], bb, gb)

            pltpu.async_copy(
                ob, out_hbm.at[pl.ds((g0 + c) * B_C, B_C)], ws)
        return carry

    lax.fori_loop(0, n_pairs, pair, 0)
    pltpu.make_async_copy(ob0, out_hbm.at[pl.ds(0, B_C)], ws0).wait()
    pltpu.make_async_copy(ob1, out_hbm.at[pl.ds(0, B_C)], ws1).wait()

  return _bond_msg_k


def _k_in_body(x_ref, w_ref, inp_ref, msg_ref):
    acc = jnp.dot(x_ref[...], w_ref[...], preferred_element_type=jnp.float32)
    inp_ref[...] = acc.astype(jnp.bfloat16)
    msg_ref[...] = jnp.maximum(acc, 0.0)


def _k_in(fb, w):
    grid = NBP // MB
    return pl.pallas_call(
        _k_in_body,
        grid=(grid,),
        in_specs=[
            pl.BlockSpec((MB, BF), lambda i: (i, 0)),
            pl.BlockSpec((BF, HP), lambda i: (0, 0)),
        ],
        out_specs=[
            pl.BlockSpec((MB, HP), lambda i: (i, 0)),
            pl.BlockSpec((MB, HP), lambda i: (i, 0)),
        ],
        out_shape=[
            jax.ShapeDtypeStruct((NBP, HP), jnp.bfloat16),
            jax.ShapeDtypeStruct((NBP, HP), jnp.float32),
        ],
    )(fb, w)


def _k_upd_body(m2_ref, inp_ref, w_ref, msg_ref):
    acc = jnp.dot(m2_ref[...].astype(jnp.bfloat16), w_ref[...],
                  preferred_element_type=jnp.float32)
    acc += inp_ref[...].astype(jnp.float32)
    msg_ref[...] = jnp.maximum(acc, 0.0)


def _k_upd(m2, inp, w):
    grid = NBP // MB
    return pl.pallas_call(
        _k_upd_body,
        grid=(grid,),
        in_specs=[
            pl.BlockSpec((MB, HP), lambda i: (i, 0)),
            pl.BlockSpec((MB, HP), lambda i: (i, 0)),
            pl.BlockSpec((HP, HP), lambda i: (0, 0)),
        ],
        out_specs=pl.BlockSpec((MB, HP), lambda i: (i, 0)),
        out_shape=jax.ShapeDtypeStruct((NBP, HP), jnp.float32),
    )(m2, inp, w)


def _k_out_body(fa_ref, am_ref, w1_ref, w2_ref, b_ref, out_ref):
    acc = jnp.dot(fa_ref[...], w1_ref[...], preferred_element_type=jnp.float32)
    acc += jnp.dot(am_ref[...].astype(jnp.bfloat16), w2_ref[...],
                   preferred_element_type=jnp.float32)
    out_ref[...] = jnp.maximum(acc + b_ref[...], 0.0)


def _k_out(fa, am, w1, w2, b):
    grid = NAP // MB
    return pl.pallas_call(
        _k_out_body,
        grid=(grid,),
        in_specs=[
            pl.BlockSpec((MB, AF), lambda i: (i, 0)),
            pl.BlockSpec((MB, HP), lambda i: (i, 0)),
            pl.BlockSpec((AF, HP), lambda i: (0, 0)),
            pl.BlockSpec((HP, HP), lambda i: (0, 0)),
            pl.BlockSpec((1, HP), lambda i: (0, 0)),
        ],
        out_specs=pl.BlockSpec((MB, HP), lambda i: (i, 0)),
        out_shape=jax.ShapeDtypeStruct((NAP, HP), jnp.float32),
    )(fa, am, w1, w2, b)


def _k_read_body(ah_ref, ids_ref, wv_ref, out_ref, sums_ref, cnts_ref):
    i = pl.program_id(0)

    @pl.when(i == 0)
    def _():
        sums_ref[...] = jnp.zeros_like(sums_ref)
        cnts_ref[...] = jnp.zeros_like(cnts_ref)

    ids = ids_ref[0]
    seg = lax.broadcasted_iota(jnp.int32, (NM, MB), 0)
    onehot = (seg == jnp.broadcast_to(ids, (NM, MB))).astype(jnp.float32)
    sums_ref[...] += jnp.dot(onehot, ah_ref[...],
                             preferred_element_type=jnp.float32)
    cnts_ref[...] += jnp.broadcast_to(
        jnp.sum(onehot, axis=1, keepdims=True), (NM, 128))

    @pl.when(i == pl.num_programs(0) - 1)
    def _():
        enc = sums_ref[...] / jnp.maximum(cnts_ref[:, 0:1], 1.0)
        score = jnp.dot(enc, wv_ref[...], preferred_element_type=jnp.float32)
        out_ref[...] = jax.nn.sigmoid(score[:NM // 2] - score[NM // 2:])


def _k_read(ah, ids3, wv):
    grid = NAP // MB
    return pl.pallas_call(
        _k_read_body,
        grid=(grid,),
        in_specs=[
            pl.BlockSpec((MB, HP), lambda i: (i, 0)),
            pl.BlockSpec((1, 1, MB), lambda i: (i, 0, 0)),
            pl.BlockSpec((HP, 1), lambda i: (0, 0)),
        ],
        out_specs=pl.BlockSpec((NM // 2, 1), lambda i: (0, 0)),
        out_shape=jax.ShapeDtypeStruct((NM // 2, 1), jnp.float32),
        scratch_shapes=[
            pltpu.VMEM((NM, HP), jnp.float32),
            pltpu.VMEM((NM, 128), jnp.float32),
        ],
    )(ah, ids3, wv)


def _k_outread_body(fa_ref, am_ref, w1_ref, w2_ref, b_ref, ids_ref, wv_ref,
                    out_ref, sums_ref, cnts_ref):
    i = pl.program_id(0)

    @pl.when(i == 0)
    def _():
        sums_ref[...] = jnp.zeros_like(sums_ref)
        cnts_ref[...] = jnp.zeros_like(cnts_ref)

    acc = jnp.dot(fa_ref[...], w1_ref[...], preferred_element_type=jnp.float32)
    acc += jnp.dot(am_ref[...].astype(jnp.bfloat16), w2_ref[...],
                   preferred_element_type=jnp.float32)
    ah = jnp.maximum(acc + b_ref[...], 0.0)

    ids = ids_ref[0]
    seg = lax.broadcasted_iota(jnp.int32, (NM, MB), 0)
    onehot = (seg == jnp.broadcast_to(ids, (NM, MB))).astype(jnp.float32)
    sums_ref[...] += jnp.dot(onehot, ah, preferred_element_type=jnp.float32)
    cnts_ref[...] += jnp.broadcast_to(
        jnp.sum(onehot, axis=1, keepdims=True), (NM, 128))

    @pl.when(i == pl.num_programs(0) - 1)
    def _():
        enc = sums_ref[...] / jnp.maximum(cnts_ref[:, 0:1], 1.0)
        score = jnp.dot(enc, wv_ref[...], preferred_element_type=jnp.float32)
        out_ref[...] = jax.nn.sigmoid(score[:NM // 2] - score[NM // 2:])


def _k_outread(fa, am, w1, w2, b, ids3, wv):
    grid = NAP // MB
    return pl.pallas_call(
        _k_outread_body,
        grid=(grid,),
        in_specs=[
            pl.BlockSpec((MB, AF), lambda i: (i, 0)),
            pl.BlockSpec((MB, HP), lambda i: (i, 0)),
            pl.BlockSpec((AF, HP), lambda i: (0, 0)),
            pl.BlockSpec((HP, HP), lambda i: (0, 0)),
            pl.BlockSpec((1, HP), lambda i: (0, 0)),
            pl.BlockSpec((1, 1, MB), lambda i: (i, 0, 0)),
            pl.BlockSpec((HP, 1), lambda i: (0, 0)),
        ],
        out_specs=pl.BlockSpec((NM // 2, 1), lambda i: (0, 0)),
        out_shape=jax.ShapeDtypeStruct((NM // 2, 1), jnp.float32),
        scratch_shapes=[
            pltpu.VMEM((NM, HP), jnp.float32),
            pltpu.VMEM((NM, 128), jnp.float32),
        ],
    )(fa, am, w1, w2, b, ids3, wv)


def kernel(f_atoms_0, f_bonds_0, a2b_0, b2a_0, b2revb_0, mol_ids_0,
           f_atoms_1, f_bonds_1, a2b_1, b2a_1, b2revb_1, mol_ids_1,
           W_i, W_h, W_o, b_o, w_ident, b_ident):
    f32 = jnp.float32
    bf16 = jnp.bfloat16

    fb = jnp.concatenate(
        [f_bonds_0.astype(bf16), f_bonds_1.astype(bf16),
         jnp.zeros((NBP - 2 * NB1, BF), bf16)], axis=0)
    fa = jnp.concatenate(
        [f_atoms_0.astype(bf16), f_atoms_1.astype(bf16),
         jnp.zeros((NAP - 2 * NA1, AF), bf16)], axis=0)
    a2b = jnp.concatenate(
        [a2b_0, a2b_1 + NB1,
         jnp.zeros((NAP - 2 * NA1, 6), jnp.int32)], axis=0)
    bpad = 16 * NC0B * B_C + 15 * NC1B * B_C + NC0B * B_C - 2 * NB1
    b2a = jnp.concatenate(
        [b2a_0, b2a_1 + NA1, jnp.zeros((bpad,), jnp.int32)], axis=0)
    b2revb = jnp.concatenate(
        [b2revb_0, b2revb_1 + NB1, jnp.zeros((bpad,), jnp.int32)], axis=0)
    mids = jnp.concatenate(
        [mol_ids_0, mol_ids_1 + NM // 2,
         jnp.full((NAP - 2 * NA1,), NM, jnp.int32)], axis=0)
    ids3 = mids.reshape(NAP // MB, 1, MB)

    idxa = (a2b.reshape(NAP // A_C, A_C, 6)
               .transpose(0, 2, 1)
               .reshape(NAP * 6))
    idxa = jnp.concatenate(
        [idxa, jnp.zeros(((NC0A - NC1A) * A_IDX,), jnp.int32)])

    wi = jnp.zeros((BF, HP), f32).at[:, :H].set(W_i).astype(bf16)
    wh = jnp.zeros((HP, HP), f32).at[:H, :H].set(W_h).astype(bf16)
    wo1 = jnp.zeros((AF, HP), f32).at[:, :H].set(W_o[:AF]).astype(bf16)
    wo2 = jnp.zeros((HP, HP), f32).at[:H, :H].set(W_o[AF:]).astype(bf16)
    bo = jnp.zeros((1, HP), f32).at[0, :H].set(b_o)
    wv = jnp.zeros((HP, 1), f32).at[:H].set(w_ident)

    inp, msg = _k_in(fb, wi)
    for _ in range(2):
        am = _nbr_sum(msg, idxa)
        m2 = _bond_msg(am, msg, b2a, b2revb)
        msg = _k_upd(m2, inp, wh)
    am = _nbr_sum(msg, idxa)
    out = _k_outread(fa, am, wo1, wo2, bo, ids3, wv)
    return out[:, 0]


# speedup vs baseline: 1.9230x; 1.1171x over previous
import functools

import jax
import jax.numpy as jnp
from jax import lax
from jax.experimental import pallas as pl
from jax.experimental.pallas import tpu as pltpu
from jax.experimental.pallas import tpu_sc as plsc

H = 300
HP = 512
PK = 256
AF = 133
BF = 147
NA1 = 10000
NB1 = 40000
NAP = 20480
NBP = 81920
NM = 512
NW = 32

MB = 640

A_C = 16
A_IDX = A_C * 6

B_C = 40

NC0A = 56
NC1A = 24
NC0B = 78
NC1B = 50


_sc_cache = {}


def _nbr_sum(msg, idxa):
    if "nbr" not in _sc_cache:
        _sc_cache["nbr"] = _build_nbr_sum()
    return _sc_cache["nbr"](msg, idxa)


def _bond_msg(am, msg, b2a, b2revb):
    if "bond" not in _sc_cache:
        _sc_cache["bond"] = _build_bond_msg()
    return _sc_cache["bond"](am, msg, b2a, b2revb)


def _sc_mesh():
    return plsc.VectorSubcoreMesh(
        core_axis_name="c", subcore_axis_name="s", num_cores=2,
        num_subcores=16)


def _row_slices():
    for u in range(PK // 16):
        yield pl.ds(u * 16, 16)


def _build_nbr_sum():
  @functools.partial(
      pl.kernel,
      out_type=jax.ShapeDtypeStruct((NAP, PK), jnp.float32),
      mesh=_sc_mesh(),
      compiler_params=pltpu.CompilerParams(needs_layout_passes=False),
      scratch_types=[
          pltpu.VMEM((NC0A * A_IDX,), jnp.int32),
          pltpu.VMEM((A_IDX, PK), jnp.float32),
          pltpu.VMEM((A_IDX, PK), jnp.float32),
          pltpu.VMEM((A_C, PK), jnp.float32),
          pltpu.VMEM((A_C, PK), jnp.float32),
          pltpu.SemaphoreType.DMA,
          pltpu.SemaphoreType.DMA,
          pltpu.SemaphoreType.DMA,
          pltpu.SemaphoreType.DMA,
      ],
  )
  def _nbr_sum_k(msg_hbm, idxa_hbm, out_hbm, idx_all,
                 buf0, buf1, acc0, acc1, gs0, gs1, ws0, ws1):
    core = lax.axis_index("c")
    sub = lax.axis_index("s")
    g0 = jnp.where(core == 0, sub * NC0A, 16 * NC0A + sub * NC1A)
    n_pairs = jnp.where(core == 0, NC0A // 2, NC1A // 2)
    n_ch = 2 * n_pairs
    pltpu.sync_copy(
        idxa_hbm.at[pl.ds(g0 * A_IDX, NC0A * A_IDX)], idx_all)
    pltpu.async_copy(
        msg_hbm.at[idx_all.at[pl.ds(0, A_IDX)]], buf0, gs0)
    pltpu.async_copy(
        msg_hbm.at[idx_all.at[pl.ds(A_IDX, A_IDX)]], buf1, gs1)

    def pair(c0, carry):
        for b, (buf, acc, gs, ws) in enumerate(
                ((buf0, acc0, gs0, ws0), (buf1, acc1, gs1, ws1))):
            c = 2 * c0 + b
            pltpu.make_async_copy(
                msg_hbm.at[idx_all.at[pl.ds(0, A_IDX)]], buf, gs).wait()

            @pl.when(c0 > 0)
            def _():
                pltpu.make_async_copy(
                    acc, out_hbm.at[pl.ds(0, A_C)], ws).wait()

            def body(a, carry2):
                for s in _row_slices():
                    v = plsc.bitcast(buf[a, s], jnp.bfloat16)
                    for k in range(1, 6):
                        v = v + plsc.bitcast(buf[k * A_C + a, s],
                                             jnp.bfloat16)
                    acc[a, s] = plsc.bitcast(v, jnp.float32)
                return carry2

            lax.fori_loop(0, A_C, body, 0)
            pltpu.async_copy(
                acc, out_hbm.at[pl.ds((g0 + c) * A_C, A_C)], ws)

            @pl.when(c + 2 < n_ch)
            def _():
                pltpu.async_copy(
                    msg_hbm.at[idx_all.at[pl.ds((c + 2) * A_IDX, A_IDX)]],
                    buf, gs)
        return carry

    lax.fori_loop(0, n_pairs, pair, 0)
    pltpu.make_async_copy(acc0, out_hbm.at[pl.ds(0, A_C)], ws0).wait()
    pltpu.make_async_copy(acc1, out_hbm.at[pl.ds(0, A_C)], ws1).wait()

  return _nbr_sum_k


def _build_bond_msg():
  @functools.partial(
      pl.kernel,
      out_type=jax.ShapeDtypeStruct((NBP, PK), jnp.float32),
      mesh=_sc_mesh(),
      compiler_params=pltpu.CompilerParams(needs_layout_passes=False),
      scratch_types=[
          pltpu.VMEM((NC0B * B_C,), jnp.int32),
          pltpu.VMEM((NC0B * B_C,), jnp.int32),
          pltpu.VMEM((B_C, PK), jnp.float32),
          pltpu.VMEM((B_C, PK), jnp.float32),
          pltpu.VMEM((B_C, PK), jnp.float32),
          pltpu.VMEM((B_C, PK), jnp.float32),
          pltpu.VMEM((B_C, PK), jnp.float32),
          pltpu.VMEM((B_C, PK), jnp.float32),
          pltpu.SemaphoreType.DMA,
          pltpu.SemaphoreType.DMA,
          pltpu.SemaphoreType.DMA,
          pltpu.SemaphoreType.DMA,
          pltpu.SemaphoreType.DMA,
          pltpu.SemaphoreType.DMA,
      ],
  )
  def _bond_msg_k(am_hbm, msg_hbm, b2a_hbm, b2revb_hbm, out_hbm,
                  i1_all, i2_all, ba0, ba1, bb0, bb1, ob0, ob1,
                  ga0, ga1, gb0, gb1, ws0, ws1):
    core = lax.axis_index("c")
    sub = lax.axis_index("s")
    g0 = jnp.where(core == 0, sub * NC0B, 16 * NC0B + sub * NC1B)
    n_pairs = jnp.where(core == 0, NC0B // 2, NC1B // 2)
    n_ch = 2 * n_pairs
    pltpu.sync_copy(b2a_hbm.at[pl.ds(g0 * B_C, NC0B * B_C)], i1_all)
    pltpu.sync_copy(b2revb_hbm.at[pl.ds(g0 * B_C, NC0B * B_C)], i2_all)
    pltpu.async_copy(am_hbm.at[i1_all.at[pl.ds(0, B_C)]], ba0, ga0)
    pltpu.async_copy(msg_hbm.at[i2_all.at[pl.ds(0, B_C)]], bb0, gb0)
    pltpu.async_copy(am_hbm.at[i1_all.at[pl.ds(B_C, B_C)]], ba1, ga1)
    pltpu.async_copy(msg_hbm.at[i2_all.at[pl.ds(B_C, B_C)]], bb1, gb1)

    def pair(c0, carry):
        for b, (ba, bb, ob, ga, gb, ws) in enumerate(
                ((ba0, bb0, ob0, ga0, gb0, ws0),
                 (ba1, bb1, ob1, ga1, gb1, ws1))):
            c = 2 * c0 + b
            pltpu.make_async_copy(
                am_hbm.at[i1_all.at[pl.ds(0, B_C)]], ba, ga).wait()
            pltpu.make_async_copy(
                msg_hbm.at[i2_all.at[pl.ds(0, B_C)]], bb, gb).wait()

            @pl.when(c0 > 0)
            def _():
                pltpu.make_async_copy(
                    ob, out_hbm.at[pl.ds(0, B_C)], ws).wait()

            def body(i, carry2):
                for s in _row_slices():
                    v = (plsc.bitcast(ba[i, s], jnp.bfloat16)
                         - plsc.bitcast(bb[i, s], jnp.bfloat16))
                    ob[i, s] = plsc.bitcast(v, jnp.float32)
                return carry2

            lax.fori_loop(0, B_C, body, 0)

            @pl.when(c + 2 < n_ch)
            def _():
                pltpu.async_copy(
                    am_hbm.at[i1_all.at[pl.ds((c + 2) * B_C, B_C)]], ba, ga)
                pltpu.async_copy(
                    msg_hbm.at[i2_all.at[pl.ds((c + 2) * B_C, B_C)]], bb, gb)

            pltpu.async_copy(
                ob, out_hbm.at[pl.ds((g0 + c) * B_C, B_C)], ws)
        return carry

    lax.fori_loop(0, n_pairs, pair, 0)
    pltpu.make_async_copy(ob0, out_hbm.at[pl.ds(0, B_C)], ws0).wait()
    pltpu.make_async_copy(ob1, out_hbm.at[pl.ds(0, B_C)], ws1).wait()

  return _bond_msg_k


def _pack(x):
    au = lax.bitcast_convert_type(x[:, :PK], jnp.uint32)
    bu = lax.bitcast_convert_type(x[:, PK:], jnp.uint32)
    au = au + jnp.uint32(0x7FFF) + ((au >> 16) & jnp.uint32(1))
    bu = bu + jnp.uint32(0x7FFF) + ((bu >> 16) & jnp.uint32(1))
    w = (au & jnp.uint32(0xFFFF0000)) | (bu >> 16)
    return lax.bitcast_convert_type(w, jnp.float32)


def _unpack(p):
    w = lax.bitcast_convert_type(p, jnp.uint32)
    a = lax.bitcast_convert_type(w & jnp.uint32(0xFFFF0000), jnp.float32)
    b = lax.bitcast_convert_type(w << 16, jnp.float32)
    return jnp.concatenate([a, b], axis=1)


def _k_in_body(x_ref, w_ref, inp_ref, msg_ref):
    acc = jnp.dot(x_ref[...], w_ref[...], preferred_element_type=jnp.float32)
    inp_ref[...] = acc.astype(jnp.bfloat16)
    msg_ref[...] = _pack(jnp.maximum(acc, 0.0))


def _k_in(fb, w):
    grid = NBP // MB
    return pl.pallas_call(
        _k_in_body,
        grid=(grid,),
        in_specs=[
            pl.BlockSpec((MB, BF), lambda i: (i, 0)),
            pl.BlockSpec((BF, HP), lambda i: (0, 0)),
        ],
        out_specs=[
            pl.BlockSpec((MB, HP), lambda i: (i, 0)),
            pl.BlockSpec((MB, PK), lambda i: (i, 0)),
        ],
        out_shape=[
            jax.ShapeDtypeStruct((NBP, HP), jnp.bfloat16),
            jax.ShapeDtypeStruct((NBP, PK), jnp.float32),
        ],
    )(fb, w)


def _k_upd_body(m2_ref, inp_ref, w_ref, msg_ref):
    x = _unpack(m2_ref[...]).astype(jnp.bfloat16)
    acc = jnp.dot(x, w_ref[...], preferred_element_type=jnp.float32)
    acc += inp_ref[...].astype(jnp.float32)
    msg_ref[...] = _pack(jnp.maximum(acc, 0.0))


def _k_upd(m2, inp, w3):
    grid = NBP // MB
    return pl.pallas_call(
        _k_upd_body,
        grid=(grid,),
        in_specs=[
            pl.BlockSpec((MB, PK), lambda i: (i, 0)),
            pl.BlockSpec((MB, HP), lambda i: (i, 0)),
            pl.BlockSpec((HP, HP), lambda i: (0, 0)),
        ],
        out_specs=pl.BlockSpec((MB, PK), lambda i: (i, 0)),
        out_shape=jax.ShapeDtypeStruct((NBP, PK), jnp.float32),
    )(m2, inp, w3)


def _k_outread_body(fa_ref, am_ref, w1_ref, w2_ref, b_ref, ids_ref, wv_ref,
                    out_ref, sums_ref, cnts_ref):
    i = pl.program_id(0)

    @pl.when(i == 0)
    def _():
        sums_ref[...] = jnp.zeros_like(sums_ref)
        cnts_ref[...] = jnp.zeros_like(cnts_ref)

    acc = jnp.dot(fa_ref[...], w1_ref[...], preferred_element_type=jnp.float32)
    acc += jnp.dot(_unpack(am_ref[...]).astype(jnp.bfloat16), w2_ref[...],
                   preferred_element_type=jnp.float32)
    ah = jnp.maximum(acc + b_ref[...], 0.0)

    ids = ids_ref[0]
    seg = lax.broadcasted_iota(jnp.int32, (NM, MB), 0)
    onehot = (seg == jnp.broadcast_to(ids, (NM, MB))).astype(jnp.float32)
    sums_ref[...] += jnp.dot(onehot, ah, preferred_element_type=jnp.float32)
    cnts_ref[...] += jnp.broadcast_to(
        jnp.sum(onehot, axis=1, keepdims=True), (NM, 128))

    @pl.when(i == pl.num_programs(0) - 1)
    def _():
        enc = sums_ref[...] / jnp.maximum(cnts_ref[:, 0:1], 1.0)
        score = jnp.dot(enc, wv_ref[...], preferred_element_type=jnp.float32)
        out_ref[...] = jax.nn.sigmoid(score[:NM // 2] - score[NM // 2:])


def _k_outread(fa, am, w1, w2, b, ids3, wv):
    grid = NAP // MB
    return pl.pallas_call(
        _k_outread_body,
        grid=(grid,),
        in_specs=[
            pl.BlockSpec((MB, AF), lambda i: (i, 0)),
            pl.BlockSpec((MB, PK), lambda i: (i, 0)),
            pl.BlockSpec((AF, HP), lambda i: (0, 0)),
            pl.BlockSpec((HP, HP), lambda i: (0, 0)),
            pl.BlockSpec((1, HP), lambda i: (0, 0)),
            pl.BlockSpec((1, 1, MB), lambda i: (i, 0, 0)),
            pl.BlockSpec((HP, 1), lambda i: (0, 0)),
        ],
        out_specs=pl.BlockSpec((NM // 2, 1), lambda i: (0, 0)),
        out_shape=jax.ShapeDtypeStruct((NM // 2, 1), jnp.float32),
        scratch_shapes=[
            pltpu.VMEM((NM, HP), jnp.float32),
            pltpu.VMEM((NM, 128), jnp.float32),
        ],
    )(fa, am, w1, w2, b, ids3, wv)


def kernel(f_atoms_0, f_bonds_0, a2b_0, b2a_0, b2revb_0, mol_ids_0,
           f_atoms_1, f_bonds_1, a2b_1, b2a_1, b2revb_1, mol_ids_1,
           W_i, W_h, W_o, b_o, w_ident, b_ident):
    f32 = jnp.float32
    bf16 = jnp.bfloat16

    fb = jnp.concatenate(
        [f_bonds_0.astype(bf16), f_bonds_1.astype(bf16),
         jnp.zeros((NBP - 2 * NB1, BF), bf16)], axis=0)
    fa = jnp.concatenate(
        [f_atoms_0.astype(bf16), f_atoms_1.astype(bf16),
         jnp.zeros((NAP - 2 * NA1, AF), bf16)], axis=0)
    a2b = jnp.concatenate(
        [a2b_0, a2b_1 + NB1,
         jnp.zeros((NAP - 2 * NA1, 6), jnp.int32)], axis=0)
    bpad = 16 * NC0B * B_C + 15 * NC1B * B_C + NC0B * B_C - 2 * NB1
    b2a = jnp.concatenate(
        [b2a_0, b2a_1 + NA1, jnp.zeros((bpad,), jnp.int32)], axis=0)
    b2revb = jnp.concatenate(
        [b2revb_0, b2revb_1 + NB1, jnp.zeros((bpad,), jnp.int32)], axis=0)
    mids = jnp.concatenate(
        [mol_ids_0, mol_ids_1 + NM // 2,
         jnp.full((NAP - 2 * NA1,), NM, jnp.int32)], axis=0)
    ids3 = mids.reshape(NAP // MB, 1, MB)

    idxa = (a2b.reshape(NAP // A_C, A_C, 6)
               .transpose(0, 2, 1)
               .reshape(NAP * 6))
    idxa = jnp.concatenate(
        [idxa, jnp.zeros(((NC0A - NC1A) * A_IDX,), jnp.int32)])

    wi = jnp.zeros((BF, HP), f32).at[:, :H].set(W_i).astype(bf16)
    wh = jnp.zeros((HP, HP), f32).at[:H, :H].set(W_h).astype(bf16)
    wo1 = jnp.zeros((AF, HP), f32).at[:, :H].set(W_o[:AF]).astype(bf16)
    wo2 = jnp.zeros((HP, HP), f32).at[:H, :H].set(W_o[AF:]).astype(bf16)
    bo = jnp.zeros((1, HP), f32).at[0, :H].set(b_o)
    wv = jnp.zeros((HP, 1), f32).at[:H].set(w_ident)

    inp, msg = _k_in(fb, wi)
    for _ in range(2):
        am = _nbr_sum(msg, idxa)
        m2 = _bond_msg(am, msg, b2a, b2revb)
        msg = _k_upd(m2, inp, wh)
    am = _nbr_sum(msg, idxa)
    out = _k_outread(fa, am, wo1, wo2, bo, ids3, wv)
    return out[:, 0]


# speedup vs baseline: 1.9292x; 1.0032x over previous
import functools

import jax
import jax.numpy as jnp
from jax import lax
from jax.experimental import pallas as pl
from jax.experimental.pallas import tpu as pltpu
from jax.experimental.pallas import tpu_sc as plsc

H = 300
HP = 512
PK = 256
AF = 133
BF = 147
NA1 = 10000
NB1 = 40000
NAP = 20480
NBP = 81920
NM = 512
NW = 32

MB = 640

A_C = 16
A_IDX = A_C * 6

B_C = 40

NC0A = 56
NC1A = 24
NC0B = 78
NC1B = 50


_sc_cache = {}


def _nbr_sum(msg, idxa):
    if "nbr" not in _sc_cache:
        _sc_cache["nbr"] = _build_nbr_sum()
    return _sc_cache["nbr"](msg, idxa)


def _bond_msg(am, msg, b2a, b2revb):
    if "bond" not in _sc_cache:
        _sc_cache["bond"] = _build_bond_msg()
    return _sc_cache["bond"](am, msg, b2a, b2revb)


def _sc_mesh():
    return plsc.VectorSubcoreMesh(
        core_axis_name="c", subcore_axis_name="s", num_cores=2,
        num_subcores=16)


def _row_slices():
    for u in range(PK // 16):
        yield pl.ds(u * 16, 16)


def _build_nbr_sum():
  @functools.partial(
      pl.kernel,
      out_type=jax.ShapeDtypeStruct((NAP, PK), jnp.float32),
      mesh=_sc_mesh(),
      compiler_params=pltpu.CompilerParams(needs_layout_passes=False),
      scratch_types=[
          pltpu.VMEM((NC0A * A_IDX,), jnp.int32),
          pltpu.VMEM((A_IDX, PK), jnp.float32),
          pltpu.VMEM((A_IDX, PK), jnp.float32),
          pltpu.VMEM((A_C, PK), jnp.float32),
          pltpu.VMEM((A_C, PK), jnp.float32),
          pltpu.SemaphoreType.DMA,
          pltpu.SemaphoreType.DMA,
          pltpu.SemaphoreType.DMA,
          pltpu.SemaphoreType.DMA,
      ],
  )
  def _nbr_sum_k(msg_hbm, idxa_hbm, out_hbm, idx_all,
                 buf0, buf1, acc0, acc1, gs0, gs1, ws0, ws1):
    core = lax.axis_index("c")
    sub = lax.axis_index("s")
    g0 = jnp.where(core == 0, sub * NC0A, 16 * NC0A + sub * NC1A)
    n_pairs = jnp.where(core == 0, NC0A // 2, NC1A // 2)
    n_ch = 2 * n_pairs
    pltpu.sync_copy(
        idxa_hbm.at[pl.ds(g0 * A_IDX, NC0A * A_IDX)], idx_all)
    pltpu.async_copy(
        msg_hbm.at[idx_all.at[pl.ds(0, A_IDX)]], buf0, gs0)
    pltpu.async_copy(
        msg_hbm.at[idx_all.at[pl.ds(A_IDX, A_IDX)]], buf1, gs1)

    def pair(c0, carry):
        for b, (buf, acc, gs, ws) in enumerate(
                ((buf0, acc0, gs0, ws0), (buf1, acc1, gs1, ws1))):
            c = 2 * c0 + b
            pltpu.make_async_copy(
                msg_hbm.at[idx_all.at[pl.ds(0, A_IDX)]], buf, gs).wait()

            @pl.when(c0 > 0)
            def _():
                pltpu.make_async_copy(
                    acc, out_hbm.at[pl.ds(0, A_C)], ws).wait()

            def body(a, carry2):
                for s in _row_slices():
                    a0, b0 = plsc.unpack(
                        plsc.bitcast(buf[a, s], jnp.bfloat16),
                        format=plsc.PackFormat.INTERLEAVED)
                    for k in range(1, 6):
                        ak, bk = plsc.unpack(
                            plsc.bitcast(buf[k * A_C + a, s], jnp.bfloat16),
                            format=plsc.PackFormat.INTERLEAVED)
                        a0 = a0 + ak
                        b0 = b0 + bk
                    acc[a, s] = plsc.bitcast(
                        plsc.pack(a0, b0,
                                  format=plsc.PackFormat.INTERLEAVED),
                        jnp.float32)
                return carry2

            lax.fori_loop(0, A_C, body, 0)
            pltpu.async_copy(
                acc, out_hbm.at[pl.ds((g0 + c) * A_C, A_C)], ws)

            @pl.when(c + 2 < n_ch)
            def _():
                pltpu.async_copy(
                    msg_hbm.at[idx_all.at[pl.ds((c + 2) * A_IDX, A_IDX)]],
                    buf, gs)
        return carry

    lax.fori_loop(0, n_pairs, pair, 0)
    pltpu.make_async_copy(acc0, out_hbm.at[pl.ds(0, A_C)], ws0).wait()
    pltpu.make_async_copy(acc1, out_hbm.at[pl.ds(0, A_C)], ws1).wait()

  return _nbr_sum_k


def _build_bond_msg():
  @functools.partial(
      pl.kernel,
      out_type=jax.ShapeDtypeStruct((NBP, PK), jnp.float32),
      mesh=_sc_mesh(),
      compiler_params=pltpu.CompilerParams(needs_layout_passes=False),
      scratch_types=[
          pltpu.VMEM((NC0B * B_C,), jnp.int32),
          pltpu.VMEM((NC0B * B_C,), jnp.int32),
          pltpu.VMEM((B_C, PK), jnp.float32),
          pltpu.VMEM((B_C, PK), jnp.float32),
          pltpu.VMEM((B_C, PK), jnp.float32),
          pltpu.VMEM((B_C, PK), jnp.float32),
          pltpu.VMEM((B_C, PK), jnp.float32),
          pltpu.VMEM((B_C, PK), jnp.float32),
          pltpu.SemaphoreType.DMA,
          pltpu.SemaphoreType.DMA,
          pltpu.SemaphoreType.DMA,
          pltpu.SemaphoreType.DMA,
          pltpu.SemaphoreType.DMA,
          pltpu.SemaphoreType.DMA,
      ],
  )
  def _bond_msg_k(am_hbm, msg_hbm, b2a_hbm, b2revb_hbm, out_hbm,
                  i1_all, i2_all, ba0, ba1, bb0, bb1, ob0, ob1,
                  ga0, ga1, gb0, gb1, ws0, ws1):
    core = lax.axis_index("c")
    sub = lax.axis_index("s")
    g0 = jnp.where(core == 0, sub * NC0B, 16 * NC0B + sub * NC1B)
    n_pairs = jnp.where(core == 0, NC0B // 2, NC1B // 2)
    n_ch = 2 * n_pairs
    pltpu.sync_copy(b2a_hbm.at[pl.ds(g0 * B_C, NC0B * B_C)], i1_all)
    pltpu.sync_copy(b2revb_hbm.at[pl.ds(g0 * B_C, NC0B * B_C)], i2_all)
    pltpu.async_copy(am_hbm.at[i1_all.at[pl.ds(0, B_C)]], ba0, ga0)
    pltpu.async_copy(msg_hbm.at[i2_all.at[pl.ds(0, B_C)]], bb0, gb0)
    pltpu.async_copy(am_hbm.at[i1_all.at[pl.ds(B_C, B_C)]], ba1, ga1)
    pltpu.async_copy(msg_hbm.at[i2_all.at[pl.ds(B_C, B_C)]], bb1, gb1)

    def pair(c0, carry):
        for b, (ba, bb, ob, ga, gb, ws) in enumerate(
                ((ba0, bb0, ob0, ga0, gb0, ws0),
                 (ba1, bb1, ob1, ga1, gb1, ws1))):
            c = 2 * c0 + b
            pltpu.make_async_copy(
                am_hbm.at[i1_all.at[pl.ds(0, B_C)]], ba, ga).wait()
            pltpu.make_async_copy(
                msg_hbm.at[i2_all.at[pl.ds(0, B_C)]], bb, gb).wait()

            @pl.when(c0 > 0)
            def _():
                pltpu.make_async_copy(
                    ob, out_hbm.at[pl.ds(0, B_C)], ws).wait()

            def body(i, carry2):
                for s in _row_slices():
                    v = (plsc.bitcast(ba[i, s], jnp.bfloat16)
                         - plsc.bitcast(bb[i, s], jnp.bfloat16))
                    ob[i, s] = plsc.bitcast(v, jnp.float32)
                return carry2

            lax.fori_loop(0, B_C, body, 0)

            @pl.when(c + 2 < n_ch)
            def _():
                pltpu.async_copy(
                    am_hbm.at[i1_all.at[pl.ds((c + 2) * B_C, B_C)]], ba, ga)
                pltpu.async_copy(
                    msg_hbm.at[i2_all.at[pl.ds((c + 2) * B_C, B_C)]], bb, gb)

            pltpu.async_copy(
                ob, out_hbm.at[pl.ds((g0 + c) * B_C, B_C)], ws)
        return carry

    lax.fori_loop(0, n_pairs, pair, 0)
    pltpu.make_async_copy(ob0, out_hbm.at[pl.ds(0, B_C)], ws0).wait()
    pltpu.make_async_copy(ob1, out_hbm.at[pl.ds(0, B_C)], ws1).wait()

  return _bond_msg_k


def _pack(x):
    au = lax.bitcast_convert_type(x[:, :PK], jnp.uint32)
    bu = lax.bitcast_convert_type(x[:, PK:], jnp.uint32)
    au = au + jnp.uint32(0x7FFF) + ((au >> 16) & jnp.uint32(1))
    bu = bu + jnp.uint32(0x7FFF) + ((bu >> 16) & jnp.uint32(1))
    w = (au & jnp.uint32(0xFFFF0000)) | (bu >> 16)
    return lax.bitcast_convert_type(w, jnp.float32)


def _unpack(p):
    w = lax.bitcast_convert_type(p, jnp.uint32)
    a = lax.bitcast_convert_type(w & jnp.uint32(0xFFFF0000), jnp.float32)
    b = lax.bitcast_convert_type(w << 16, jnp.float32)
    return jnp.concatenate([a, b], axis=1)


def _k_in_body(x_ref, w_ref, inp_ref, msg_ref):
    acc = jnp.dot(x_ref[...], w_ref[...], preferred_element_type=jnp.float32)
    inp_ref[...] = acc.astype(jnp.bfloat16)
    msg_ref[...] = _pack(jnp.maximum(acc, 0.0))


def _k_in(fb, w):
    grid = NBP // MB
    return pl.pallas_call(
        _k_in_body,
        grid=(grid,),
        in_specs=[
            pl.BlockSpec((MB, BF), lambda i: (i, 0)),
            pl.BlockSpec((BF, HP), lambda i: (0, 0)),
        ],
        out_specs=[
            pl.BlockSpec((MB, HP), lambda i: (i, 0)),
            pl.BlockSpec((MB, PK), lambda i: (i, 0)),
        ],
        out_shape=[
            jax.ShapeDtypeStruct((NBP, HP), jnp.bfloat16),
            jax.ShapeDtypeStruct((NBP, PK), jnp.float32),
        ],
    )(fb, w)


def _k_upd_body(m2_ref, inp_ref, w_ref, msg_ref):
    x = _unpack(m2_ref[...]).astype(jnp.bfloat16)
    acc = jnp.dot(x, w_ref[...], preferred_element_type=jnp.float32)
    acc += inp_ref[...].astype(jnp.float32)
    msg_ref[...] = _pack(jnp.maximum(acc, 0.0))


def _k_upd(m2, inp, w3):
    grid = NBP // MB
    return pl.pallas_call(
        _k_upd_body,
        grid=(grid,),
        in_specs=[
            pl.BlockSpec((MB, PK), lambda i: (i, 0)),
            pl.BlockSpec((MB, HP), lambda i: (i, 0)),
            pl.BlockSpec((HP, HP), lambda i: (0, 0)),
        ],
        out_specs=pl.BlockSpec((MB, PK), lambda i: (i, 0)),
        out_shape=jax.ShapeDtypeStruct((NBP, PK), jnp.float32),
    )(m2, inp, w3)


def _k_outread_body(fa_ref, am_ref, w1_ref, w2_ref, b_ref, ids_ref, wv_ref,
                    out_ref, sums_ref, cnts_ref):
    i = pl.program_id(0)

    @pl.when(i == 0)
    def _():
        sums_ref[...] = jnp.zeros_like(sums_ref)
        cnts_ref[...] = jnp.zeros_like(cnts_ref)

    acc = jnp.dot(fa_ref[...], w1_ref[...], preferred_element_type=jnp.float32)
    acc += jnp.dot(_unpack(am_ref[...]).astype(jnp.bfloat16), w2_ref[...],
                   preferred_element_type=jnp.float32)
    ah = jnp.maximum(acc + b_ref[...], 0.0)

    ids = ids_ref[0]
    seg = lax.broadcasted_iota(jnp.int32, (NM, MB), 0)
    onehot = (seg == jnp.broadcast_to(ids, (NM, MB))).astype(jnp.float32)
    sums_ref[...] += jnp.dot(onehot, ah, preferred_element_type=jnp.float32)
    cnts_ref[...] += jnp.broadcast_to(
        jnp.sum(onehot, axis=1, keepdims=True), (NM, 128))

    @pl.when(i == pl.num_programs(0) - 1)
    def _():
        enc = sums_ref[...] / jnp.maximum(cnts_ref[:, 0:1], 1.0)
        score = jnp.dot(enc, wv_ref[...], preferred_element_type=jnp.float32)
        out_ref[...] = jax.nn.sigmoid(score[:NM // 2] - score[NM // 2:])


def _k_outread(fa, am, w1, w2, b, ids3, wv):
    grid = NAP // MB
    return pl.pallas_call(
        _k_outread_body,
        grid=(grid,),
        in_specs=[
            pl.BlockSpec((MB, AF), lambda i: (i, 0)),
            pl.BlockSpec((MB, PK), lambda i: (i, 0)),
            pl.BlockSpec((AF, HP), lambda i: (0, 0)),
            pl.BlockSpec((HP, HP), lambda i: (0, 0)),
            pl.BlockSpec((1, HP), lambda i: (0, 0)),
            pl.BlockSpec((1, 1, MB), lambda i: (i, 0, 0)),
            pl.BlockSpec((HP, 1), lambda i: (0, 0)),
        ],
        out_specs=pl.BlockSpec((NM // 2, 1), lambda i: (0, 0)),
        out_shape=jax.ShapeDtypeStruct((NM // 2, 1), jnp.float32),
        scratch_shapes=[
            pltpu.VMEM((NM, HP), jnp.float32),
            pltpu.VMEM((NM, 128), jnp.float32),
        ],
    )(fa, am, w1, w2, b, ids3, wv)


def kernel(f_atoms_0, f_bonds_0, a2b_0, b2a_0, b2revb_0, mol_ids_0,
           f_atoms_1, f_bonds_1, a2b_1, b2a_1, b2revb_1, mol_ids_1,
           W_i, W_h, W_o, b_o, w_ident, b_ident):
    f32 = jnp.float32
    bf16 = jnp.bfloat16

    fb = jnp.concatenate(
        [f_bonds_0.astype(bf16), f_bonds_1.astype(bf16),
         jnp.zeros((NBP - 2 * NB1, BF), bf16)], axis=0)
    fa = jnp.concatenate(
        [f_atoms_0.astype(bf16), f_atoms_1.astype(bf16),
         jnp.zeros((NAP - 2 * NA1, AF), bf16)], axis=0)
    a2b = jnp.concatenate(
        [a2b_0, a2b_1 + NB1,
         jnp.zeros((NAP - 2 * NA1, 6), jnp.int32)], axis=0)
    bpad = 16 * NC0B * B_C + 15 * NC1B * B_C + NC0B * B_C - 2 * NB1
    b2a = jnp.concatenate(
        [b2a_0, b2a_1 + NA1, jnp.zeros((bpad,), jnp.int32)], axis=0)
    b2revb = jnp.concatenate(
        [b2revb_0, b2revb_1 + NB1, jnp.zeros((bpad,), jnp.int32)], axis=0)
    mids = jnp.concatenate(
        [mol_ids_0, mol_ids_1 + NM // 2,
         jnp.full((NAP - 2 * NA1,), NM, jnp.int32)], axis=0)
    ids3 = mids.reshape(NAP // MB, 1, MB)

    idxa = (a2b.reshape(NAP // A_C, A_C, 6)
               .transpose(0, 2, 1)
               .reshape(NAP * 6))
    idxa = jnp.concatenate(
        [idxa, jnp.zeros(((NC0A - NC1A) * A_IDX,), jnp.int32)])

    wi = jnp.zeros((BF, HP), f32).at[:, :H].set(W_i).astype(bf16)
    wh = jnp.zeros((HP, HP), f32).at[:H, :H].set(W_h).astype(bf16)
    wo1 = jnp.zeros((AF, HP), f32).at[:, :H].set(W_o[:AF]).astype(bf16)
    wo2 = jnp.zeros((HP, HP), f32).at[:H, :H].set(W_o[AF:]).astype(bf16)
    bo = jnp.zeros((1, HP), f32).at[0, :H].set(b_o)
    wv = jnp.zeros((HP, 1), f32).at[:H].set(w_ident)

    inp, msg = _k_in(fb, wi)
    for _ in range(2):
        am = _nbr_sum(msg, idxa)
        m2 = _bond_msg(am, msg, b2a, b2revb)
        msg = _k_upd(m2, inp, wh)
    am = _nbr_sum(msg, idxa)
    out = _k_outread(fa, am, wo1, wo2, bo, ids3, wv)
    return out[:, 0]
